# baseline (speedup 1.0000x reference)
import functools

import jax
import jax.numpy as jnp
from jax import lax
from jax.experimental import pallas as pl
from jax.experimental.pallas import tpu as pltpu
from jax.experimental.pallas import tpu_sc as plsc

_N, _E, _S, _R = 10000, 320000, 128, 3
_NC, _NS = 2, 16
_NW = _NC * _NS
_CHUNK = 128
_CPT = 80
_HALF = 40
_NCHUNKS = _CPT * _NW
_EPAD = _NCHUNKS * _CHUNK
_NA = _N + 8
_RPT = 624


def _sc_agg_body(msg, src2, dst2, zero, out,
                 sidx, didx, rows0, rows1, acc, sem0, sem1):
    cid = lax.axis_index("c")
    sid = lax.axis_index("s")
    wid = sid * _NC + cid
    start = wid * _CPT

    pltpu.sync_copy(zero.at[pl.ds(sid * _RPT, _RPT)], acc.at[pl.ds(sid * _RPT, _RPT)])

    @pl.when(sid == _NS - 1)
    def _():
        pltpu.sync_copy(zero.at[pl.ds(_NS * _RPT, _N - _NS * _RPT)],
                        acc.at[pl.ds(_NS * _RPT, _N - _NS * _RPT)])

    plsc.subcore_barrier()

    for h in range(_CPT // _HALF):
        hbase = start + h * _HALF
        pltpu.sync_copy(src2.at[pl.ds(hbase, _HALF)], sidx)
        pltpu.sync_copy(dst2.at[pl.ds(hbase, _HALF)], didx)

        pltpu.async_copy(msg.at[sidx.at[0]], rows0, sem0)

        def body(p, carry):
            j0 = 2 * p
            pltpu.async_copy(msg.at[sidx.at[j0 + 1]], rows1, sem1)
            pltpu.make_async_copy(msg.at[sidx.at[j0]], rows0, sem0).wait()
            pltpu.sync_copy(rows0, acc.at[didx.at[j0]], add=True)

            @pl.when(j0 + 2 < _HALF)
            def _():
                pltpu.async_copy(msg.at[sidx.at[j0 + 2]], rows0, sem0)

            pltpu.make_async_copy(msg.at[sidx.at[j0 + 1]], rows1, sem1).wait()
            pltpu.sync_copy(rows1, acc.at[didx.at[j0 + 1]], add=True)
            return carry

        lax.fori_loop(0, _HALF // 2, body, 0)

    plsc.subcore_barrier()

    pltpu.sync_copy(acc.at[pl.ds(sid * _RPT, _RPT)],
                    out.at[cid].at[pl.ds(sid * _RPT, _RPT)])

    @pl.when(sid == _NS - 1)
    def _():
        pltpu.sync_copy(acc.at[pl.ds(_NS * _RPT, _N - _NS * _RPT)],
                        out.at[cid].at[pl.ds(_NS * _RPT, _N - _NS * _RPT)])


_sc_aggregate = pl.kernel(
    _sc_agg_body,
    out_type=jax.ShapeDtypeStruct((_NC, _N, _S), jnp.float32),
    mesh=plsc.VectorSubcoreMesh(core_axis_name="c", subcore_axis_name="s"),
    scratch_types=[
        pltpu.VMEM((_HALF, _CHUNK), jnp.int32),
        pltpu.VMEM((_HALF, _CHUNK), jnp.int32),
        pltpu.VMEM((_CHUNK, _S), jnp.float32),
        pltpu.VMEM((_CHUNK, _S), jnp.float32),
        pltpu.VMEM_SHARED((_NA, _S), jnp.float32),
        pltpu.SemaphoreType.DMA,
        pltpu.SemaphoreType.DMA,
    ],
)


_BM = 2000


def _mm_relu_body(x_ref, w_ref, b_ref, o_ref):
    o_ref[...] = jnp.maximum(
        jnp.dot(x_ref[...], w_ref[...], preferred_element_type=jnp.float32)
        + b_ref[...], 0.0)


def _mm_relu(x, w, b):
    n, d = x.shape
    s = w.shape[1]
    return pl.pallas_call(
        _mm_relu_body,
        grid=(n // _BM,),
        in_specs=[
            pl.BlockSpec((_BM, d), lambda i: (i, 0)),
            pl.BlockSpec((d, s), lambda i: (0, 0)),
            pl.BlockSpec((1, s), lambda i: (0, 0)),
        ],
        out_specs=pl.BlockSpec((_BM, s), lambda i: (i, 0)),
        out_shape=jax.ShapeDtypeStruct((n, s), jnp.float32),
    )(x, w, b.reshape(1, s))


def _update_body(s_ref, p_ref, w_ref, b_ref, o_ref):
    agg = p_ref[0] + p_ref[1]
    o_ref[...] = s_ref[...] + jnp.maximum(
        jnp.dot(agg, w_ref[...], preferred_element_type=jnp.float32)
        + b_ref[...], 0.0)


def _update(state, partials, w, b):
    return pl.pallas_call(
        _update_body,
        grid=(_N // _BM,),
        in_specs=[
            pl.BlockSpec((_BM, _S), lambda i: (i, 0)),
            pl.BlockSpec((_NC, _BM, _S), lambda i: (0, i, 0)),
            pl.BlockSpec((_S, _S), lambda i: (0, 0)),
            pl.BlockSpec((1, _S), lambda i: (0, 0)),
        ],
        out_specs=pl.BlockSpec((_BM, _S), lambda i: (i, 0)),
        out_shape=jax.ShapeDtypeStruct((_N, _S), jnp.float32),
    )(state, partials, w, b.reshape(1, _S))


def _head_body(s_ref, w1_ref, b1_ref, w2_ref, b2_ref, o_ref):
    h = jnp.maximum(
        jnp.dot(s_ref[...], w1_ref[...], preferred_element_type=jnp.float32)
        + b1_ref[...], 0.0)
    o = jnp.dot(h, w2_ref[...], preferred_element_type=jnp.float32) + b2_ref[...]
    mean = o[:, 0:1]
    scale = jnp.exp(o[:, 1:2])
    o_ref[...] = jnp.concatenate([mean, scale], axis=1)


def _head(state, w1, b1, w2, b2):
    return pl.pallas_call(
        _head_body,
        grid=(_N // _BM,),
        in_specs=[
            pl.BlockSpec((_BM, _S), lambda i: (i, 0)),
            pl.BlockSpec((_S, _S), lambda i: (0, 0)),
            pl.BlockSpec((1, _S), lambda i: (0, 0)),
            pl.BlockSpec((_S, 2), lambda i: (0, 0)),
            pl.BlockSpec((1, 2), lambda i: (0, 0)),
        ],
        out_specs=pl.BlockSpec((_BM, 2), lambda i: (i, 0)),
        out_shape=jax.ShapeDtypeStruct((_N, 2), jnp.float32),
    )(state, w1, b1.reshape(1, _S), w2, b2.reshape(1, 2))


def kernel(x, edge_index, batch, W_in, b_in, Wm, bm, Wu, bu, Wo1, bo1, Wo2, bo2):
    pad = _EPAD - _E
    src = jnp.concatenate(
        [edge_index[0], jnp.zeros((pad,), jnp.int32)]).reshape(_NCHUNKS, _CHUNK)
    dst = jnp.concatenate(
        [edge_index[1], jnp.full((pad,), _N, jnp.int32)]).reshape(_NCHUNKS, _CHUNK)
    zeros = jnp.zeros((_N, _S), jnp.float32)

    state = _mm_relu(x, W_in, b_in)
    for r in range(_R):
        message = _mm_relu(state, Wm[r], bm[r])
        partials = _sc_aggregate(message, src, dst, zeros)
        state = _update(state, partials, Wu[r], bu[r])

    out = _head(state, Wo1, bo1, Wo2, bo2)
    return out[:, 0:1], out[:, 1:2]


# speedup vs baseline: 3.4348x; 3.4348x over previous
import functools

import jax
import jax.numpy as jnp
from jax import lax
from jax.experimental import pallas as pl
from jax.experimental.pallas import tpu as pltpu
from jax.experimental.pallas import tpu_sc as plsc

_N, _E, _S, _R = 10000, 320000, 128, 3
_NC, _NS = 2, 16
_NW = _NC * _NS
_CHUNK = 128
_CPT = 80
_HALF = 40
_NCHUNKS = _CPT * _NW
_EPAD = _NCHUNKS * _CHUNK
_NA = _N + _CHUNK
_RPT = 624


def _sc_agg_body(msg, src2, dst2, zero, out,
                 sidx, didx, rows0, rows1, acc, sem0, sem1):
    cid = lax.axis_index("c")
    sid = lax.axis_index("s")
    wid = sid * _NC + cid
    start = wid * _CPT

    pltpu.sync_copy(zero.at[pl.ds(sid * _RPT, _RPT)], acc.at[pl.ds(sid * _RPT, _RPT)])

    @pl.when(sid == _NS - 1)
    def _():
        pltpu.sync_copy(zero.at[pl.ds(_NS * _RPT, _N - _NS * _RPT)],
                        acc.at[pl.ds(_NS * _RPT, _N - _NS * _RPT)])

    plsc.subcore_barrier()

    for h in range(_CPT // _HALF):
        hbase = start + h * _HALF
        pltpu.sync_copy(src2.at[pl.ds(hbase, _HALF)], sidx)
        pltpu.sync_copy(dst2.at[pl.ds(hbase, _HALF)], didx)

        pltpu.async_copy(msg.at[sidx.at[0]], rows0, sem0)

        def body(p, carry):
            j0 = 2 * p
            pltpu.async_copy(msg.at[sidx.at[j0 + 1]], rows1, sem1)
            pltpu.make_async_copy(msg.at[sidx.at[j0]], rows0, sem0).wait()
            pltpu.sync_copy(rows0, acc.at[didx.at[j0]], add=True)

            @pl.when(j0 + 2 < _HALF)
            def _():
                pltpu.async_copy(msg.at[sidx.at[j0 + 2]], rows0, sem0)

            pltpu.make_async_copy(msg.at[sidx.at[j0 + 1]], rows1, sem1).wait()
            pltpu.sync_copy(rows1, acc.at[didx.at[j0 + 1]], add=True)
            return carry

        lax.fori_loop(0, _HALF // 2, body, 0)

    plsc.subcore_barrier()

    pltpu.sync_copy(acc.at[pl.ds(sid * _RPT, _RPT)],
                    out.at[cid].at[pl.ds(sid * _RPT, _RPT)])

    @pl.when(sid == _NS - 1)
    def _():
        pltpu.sync_copy(acc.at[pl.ds(_NS * _RPT, _N - _NS * _RPT)],
                        out.at[cid].at[pl.ds(_NS * _RPT, _N - _NS * _RPT)])


_sc_aggregate = pl.kernel(
    _sc_agg_body,
    out_type=jax.ShapeDtypeStruct((_NC, _N, _S), jnp.float32),
    mesh=plsc.VectorSubcoreMesh(core_axis_name="c", subcore_axis_name="s"),
    scratch_types=[
        pltpu.VMEM((_HALF, _CHUNK), jnp.int32),
        pltpu.VMEM((_HALF, _CHUNK), jnp.int32),
        pltpu.VMEM((_CHUNK, _S), jnp.float32),
        pltpu.VMEM((_CHUNK, _S), jnp.float32),
        pltpu.VMEM_SHARED((_NA, _S), jnp.float32),
        pltpu.SemaphoreType.DMA,
        pltpu.SemaphoreType.DMA,
    ],
)


_BM = 2000


def _mm_relu_body(x_ref, w_ref, b_ref, o_ref):
    o_ref[...] = jnp.maximum(
        jnp.dot(x_ref[...], w_ref[...], preferred_element_type=jnp.float32)
        + b_ref[...], 0.0)


def _mm_relu(x, w, b):
    n, d = x.shape
    s = w.shape[1]
    return pl.pallas_call(
        _mm_relu_body,
        grid=(n // _BM,),
        in_specs=[
            pl.BlockSpec((_BM, d), lambda i: (i, 0)),
            pl.BlockSpec((d, s), lambda i: (0, 0)),
            pl.BlockSpec((1, s), lambda i: (0, 0)),
        ],
        out_specs=pl.BlockSpec((_BM, s), lambda i: (i, 0)),
        out_shape=jax.ShapeDtypeStruct((n, s), jnp.float32),
    )(x, w, b.reshape(1, s))


def _update_body(s_ref, p_ref, w_ref, b_ref, o_ref):
    agg = p_ref[0] + p_ref[1]
    o_ref[...] = s_ref[...] + jnp.maximum(
        jnp.dot(agg, w_ref[...], preferred_element_type=jnp.float32)
        + b_ref[...], 0.0)


def _update(state, partials, w, b):
    return pl.pallas_call(
        _update_body,
        grid=(_N // _BM,),
        in_specs=[
            pl.BlockSpec((_BM, _S), lambda i: (i, 0)),
            pl.BlockSpec((_NC, _BM, _S), lambda i: (0, i, 0)),
            pl.BlockSpec((_S, _S), lambda i: (0, 0)),
            pl.BlockSpec((1, _S), lambda i: (0, 0)),
        ],
        out_specs=pl.BlockSpec((_BM, _S), lambda i: (i, 0)),
        out_shape=jax.ShapeDtypeStruct((_N, _S), jnp.float32),
    )(state, partials, w, b.reshape(1, _S))


def _head_body(s_ref, w1_ref, b1_ref, w2_ref, b2_ref, o_ref):
    h = jnp.maximum(
        jnp.dot(s_ref[...], w1_ref[...], preferred_element_type=jnp.float32)
        + b1_ref[...], 0.0)
    o = jnp.dot(h, w2_ref[...], preferred_element_type=jnp.float32) + b2_ref[...]
    mean = o[:, 0:1]
    scale = jnp.exp(o[:, 1:2])
    o_ref[...] = jnp.concatenate([mean, scale], axis=1)


def _head(state, w1, b1, w2, b2):
    return pl.pallas_call(
        _head_body,
        grid=(_N // _BM,),
        in_specs=[
            pl.BlockSpec((_BM, _S), lambda i: (i, 0)),
            pl.BlockSpec((_S, _S), lambda i: (0, 0)),
            pl.BlockSpec((1, _S), lambda i: (0, 0)),
            pl.BlockSpec((_S, 2), lambda i: (0, 0)),
            pl.BlockSpec((1, 2), lambda i: (0, 0)),
        ],
        out_specs=pl.BlockSpec((_BM, 2), lambda i: (i, 0)),
        out_shape=jax.ShapeDtypeStruct((_N, 2), jnp.float32),
    )(state, w1, b1.reshape(1, _S), w2, b2.reshape(1, 2))


def kernel(x, edge_index, batch, W_in, b_in, Wm, bm, Wu, bu, Wo1, bo1, Wo2, bo2):
    pad = _EPAD - _E
    lanes = jnp.arange(pad, dtype=jnp.int32) % _CHUNK
    src = jnp.concatenate(
        [edge_index[0], lanes]).reshape(_NCHUNKS, _CHUNK)
    dst = jnp.concatenate(
        [edge_index[1], _N + lanes]).reshape(_NCHUNKS, _CHUNK)
    zeros = jnp.zeros((_N, _S), jnp.float32)

    state = _mm_relu(x, W_in, b_in)
    for r in range(_R):
        message = _mm_relu(state, Wm[r], bm[r])
        partials = _sc_aggregate(message, src, dst, zeros)
        state = _update(state, partials, Wu[r], bu[r])

    out = _head(state, Wo1, bo1, Wo2, bo2)
    return out[:, 0:1], out[:, 1:2]
